# Initial kernel scaffold; baseline (speedup 1.0000x reference)
import jax
import jax.numpy as jnp
from jax.experimental import pallas as pl


def kernel(latents, embedding_weight):
    raise NotImplementedError("write your pallas kernel here")


# speedup vs baseline: 1.0241x; 1.0241x over previous
import functools

import jax
import jax.numpy as jnp
from jax import lax
from jax.experimental import pallas as pl
from jax.experimental.pallas import tpu as pltpu
from jax.experimental.pallas import tpu_sc as plsc

_K = 8192
_D = 256
_N = 8192
_TOK_BLK = 1024
_COMMIT = 0.25

_NCORES = 2
_NSUB = 16
_NW = _NCORES * _NSUB
_CHUNK = 128
_CPW = _N // (_NW * _CHUNK)


def _gather_body(table_hbm, idx_hbm, out_hbm, idx_v, rows_v, sem):
    wid = lax.axis_index("s") * _NCORES + lax.axis_index("c")
    base = wid * _CPW
    pltpu.sync_copy(idx_hbm.at[pl.ds(base, _CPW)], idx_v)
    copies = [
        pltpu.async_copy(table_hbm.at[idx_v.at[j]], rows_v.at[j], sem)
        for j in range(_CPW)
    ]
    for c in copies:
        c.wait()
    pltpu.sync_copy(rows_v, out_hbm.at[pl.ds(base, _CPW)])


@functools.cache
def _sc_gather():
    return pl.kernel(
        _gather_body,
        out_type=jax.ShapeDtypeStruct((_N // _CHUNK, _CHUNK, _D), jnp.float32),
        mesh=plsc.VectorSubcoreMesh(core_axis_name="c", subcore_axis_name="s"),
        scratch_types=[
            pltpu.VMEM((_CPW, _CHUNK), jnp.int32),
            pltpu.VMEM((_CPW, _CHUNK, _D), jnp.float32),
            pltpu.SemaphoreType.DMA,
        ],
    )


def _loss_xpose_body(q_ref, x_ref, qt_ref, loss_ref):
    q = q_ref[...]
    x = x_ref[...]
    diff = q - x
    loss_ref[...] = jnp.sum(diff * diff).reshape(1, 1, 1)
    qt_ref[...] = q.T[None]


def _loss_xpose_call(qflat, flat):
    n_blocks = _N // _TOK_BLK
    return pl.pallas_call(
        _loss_xpose_body,
        grid=(n_blocks,),
        in_specs=[
            pl.BlockSpec((_TOK_BLK, _D), lambda i: (i, 0)),
            pl.BlockSpec((_TOK_BLK, _D), lambda i: (i, 0)),
        ],
        out_specs=[
            pl.BlockSpec((1, _D, _TOK_BLK), lambda i: (i, 0, 0)),
            pl.BlockSpec((1, 1, 1), lambda i: (i, 0, 0)),
        ],
        out_shape=[
            jax.ShapeDtypeStruct((n_blocks, _D, _TOK_BLK), jnp.float32),
            jax.ShapeDtypeStruct((n_blocks, 1, 1), jnp.float32),
        ],
    )(qflat, flat)


def kernel(latents, embedding_weight):
    b, c, h, w = latents.shape
    lat = jnp.transpose(latents, (0, 2, 3, 1))
    flat = lat.reshape(-1, _D)
    distances = (jnp.sum(flat ** 2, axis=1, keepdims=True)
                 + jnp.sum(embedding_weight ** 2, axis=1)
                 - 2.0 * jnp.matmul(flat, embedding_weight.T))
    idx = jnp.argmin(distances, axis=1)
    qrows = _sc_gather()(embedding_weight, idx.reshape(_N // _CHUNK, _CHUNK))
    qflat = qrows.reshape(_N, _D)
    qt, loss_parts = _loss_xpose_call(qflat, flat)
    vq_loss = _COMMIT * (jnp.sum(loss_parts) / (_N * _D))
    quantized_out = qt.reshape(b, _D, h, w)
    indices_out = idx.reshape(b, h, w)
    return (vq_loss, quantized_out, indices_out)


# speedup vs baseline: 1.0248x; 1.0006x over previous
import functools

import jax
import jax.numpy as jnp
from jax import lax
from jax.experimental import pallas as pl
from jax.experimental.pallas import tpu as pltpu
from jax.experimental.pallas import tpu_sc as plsc

_K = 8192
_D = 256
_N = 8192
_TOK_BLK = 1024
_COMMIT = 0.25

_NCORES = 2
_NSUB = 16
_NW = _NCORES * _NSUB
_CHUNK = 128
_CPW = _N // (_NW * _CHUNK)


def _gather_body(table_hbm, idx_hbm, out_hbm, idx_v, rows_v, sem):
    wid = lax.axis_index("s") * _NCORES + lax.axis_index("c")
    base = wid * _CPW * _CHUNK
    pltpu.sync_copy(idx_hbm.at[pl.ds(base, _CPW * _CHUNK)], idx_v)
    copies = [
        pltpu.async_copy(table_hbm.at[idx_v.at[pl.ds(j * _CHUNK, _CHUNK)]],
                         rows_v.at[j], sem)
        for j in range(_CPW)
    ]
    for c in copies:
        c.wait()
    for j in range(_CPW):
        pltpu.sync_copy(rows_v.at[j],
                        out_hbm.at[pl.ds(base + j * _CHUNK, _CHUNK)])


@functools.cache
def _sc_gather():
    return pl.kernel(
        _gather_body,
        out_type=jax.ShapeDtypeStruct((_N, _D), jnp.float32),
        mesh=plsc.VectorSubcoreMesh(core_axis_name="c", subcore_axis_name="s"),
        scratch_types=[
            pltpu.VMEM((_CPW * _CHUNK,), jnp.int32),
            pltpu.VMEM((_CPW, _CHUNK, _D), jnp.float32),
            pltpu.SemaphoreType.DMA,
        ],
    )


def _loss_xpose_body(q_ref, x_ref, qt_ref, loss_ref):
    q = q_ref[...]
    x = x_ref[...]
    diff = q - x
    loss_ref[...] = jnp.sum(diff * diff).reshape(1, 1, 1)
    qt_ref[...] = q.T[None]


def _loss_xpose_call(qflat, flat):
    n_blocks = _N // _TOK_BLK
    return pl.pallas_call(
        _loss_xpose_body,
        grid=(n_blocks,),
        in_specs=[
            pl.BlockSpec((_TOK_BLK, _D), lambda i: (i, 0)),
            pl.BlockSpec((_TOK_BLK, _D), lambda i: (i, 0)),
        ],
        out_specs=[
            pl.BlockSpec((1, _D, _TOK_BLK), lambda i: (i, 0, 0)),
            pl.BlockSpec((1, 1, 1), lambda i: (i, 0, 0)),
        ],
        out_shape=[
            jax.ShapeDtypeStruct((n_blocks, _D, _TOK_BLK), jnp.float32),
            jax.ShapeDtypeStruct((n_blocks, 1, 1), jnp.float32),
        ],
    )(qflat, flat)


def kernel(latents, embedding_weight):
    b, c, h, w = latents.shape
    lat = jnp.transpose(latents, (0, 2, 3, 1))
    flat = lat.reshape(-1, _D)
    distances = (jnp.sum(flat ** 2, axis=1, keepdims=True)
                 + jnp.sum(embedding_weight ** 2, axis=1)
                 - 2.0 * jnp.matmul(flat, embedding_weight.T))
    idx = jnp.argmin(distances, axis=1)
    qflat = _sc_gather()(embedding_weight, idx)
    qt, loss_parts = _loss_xpose_call(qflat, flat)
    vq_loss = _COMMIT * (jnp.sum(loss_parts) / (_N * _D))
    quantized_out = qt.reshape(b, _D, h, w)
    indices_out = idx.reshape(b, h, w)
    return (vq_loss, quantized_out, indices_out)
